# baseline (speedup 1.0000x reference)
import functools

import jax
import jax.numpy as jnp
import numpy as np
from jax import lax
from jax.experimental import pallas as pl
from jax.experimental.pallas import tpu as pltpu
from jax.experimental.pallas import tpu_sc as plsc

_TEMPERATURE = 0.02
_K = 10
_NEG = float(np.finfo(np.float32).min)


def _make_sc_col_thresholds(B, k):
    info = plsc.get_sparse_core_info()
    NC, NS, L = info.num_cores, info.num_subcores, info.num_lanes
    NW = NC * NS
    groups_per_w = B // (NW * L)

    mesh = plsc.VectorSubcoreMesh(core_axis_name="c", subcore_axis_name="s")

    @functools.partial(
        pl.kernel,
        mesh=mesh,
        out_type=jax.ShapeDtypeStruct((B,), jnp.float32),
        compiler_params=pltpu.CompilerParams(use_tc_tiling_on_sc=False),
        scratch_types=[
            pltpu.VMEM((B, L), jnp.float32),
            pltpu.VMEM((L,), jnp.float32),
        ],
    )
    def sc_kernel(sim_hbm, out_hbm, colbuf, outbuf):
        wid = lax.axis_index("s") * NC + lax.axis_index("c")
        for g in range(groups_per_w):
            c0 = wid * (groups_per_w * L) + g * L
            pltpu.sync_copy(sim_hbm.at[:, pl.ds(c0, L)], colbuf)

            def body(r, carry):
                v = colbuf[r, :]
                new = []
                u = v
                for i in range(k):
                    hi = jnp.maximum(carry[i], u)
                    u = jnp.minimum(carry[i], u)
                    new.append(hi)
                return tuple(new)

            init = tuple(jnp.full((L,), _NEG, jnp.float32) for _ in range(k))
            carry = lax.fori_loop(0, B, body, init)
            outbuf[...] = carry[k - 1]
            pltpu.sync_copy(outbuf, out_hbm.at[pl.ds(c0, L)])

    return sc_kernel


def _tc1_body(row_ref, tr_ref, ent_ref, *, k: int, max_ent: float):
    X = row_ref[...]
    xm = X
    tr = None
    rowmax = None
    for it in range(k):
        tr = jnp.max(xm, axis=1, keepdims=True)
        if it == 0:
            rowmax = tr
        xm = jnp.where(xm >= tr, _NEG, xm)

    inv_t = 1.0 / _TEMPERATURE
    sm = (X - rowmax) * inv_t
    e = jnp.exp(sm)
    Z = jnp.sum(e, axis=1, keepdims=True)
    S1 = jnp.sum(sm * e, axis=1, keepdims=True)
    ent_ref[...] = (jnp.log(Z) - S1 / Z)[:, 0] * (1.0 / max_ent)
    tr_ref[...] = tr[:, 0]


def _tc2_body(row_ref, col_ref, tr_ref, tc_ref, ent_ref, unc_ref, *, k: int):
    X = row_ref[...]
    Y = col_ref[...]
    blk = X.shape[0]
    tr = tr_ref[...].reshape(blk, 1)
    tc = tc_ref[...].reshape(1, blk)
    R = (X >= tr).astype(jnp.float32)
    C = (Y >= tc).astype(jnp.float32)
    P = jax.lax.dot(R, C, preferred_element_type=jnp.float32)
    ii = jax.lax.broadcasted_iota(jnp.int32, (blk, blk), 0)
    jj = jax.lax.broadcasted_iota(jnp.int32, (blk, blk), 1)
    matches = jnp.sum(jnp.where(ii == jj, P, 0.0), axis=1)
    ra = matches * (1.0 / k)
    unc_ref[...] = (1.0 - ra) * 0.5 + ent_ref[...] * 0.5


def kernel(sim_matrix, pids):
    del pids
    B = sim_matrix.shape[0]
    blk = 512
    k = min(_K, B)
    max_ent = float(np.log(B + 1e-10))
    grid = B // blk

    t_col = _make_sc_col_thresholds(B, k)(sim_matrix)

    t_row, ent = pl.pallas_call(
        functools.partial(_tc1_body, k=k, max_ent=max_ent),
        grid=(grid,),
        in_specs=[pl.BlockSpec((blk, B), lambda i: (i, 0))],
        out_specs=[
            pl.BlockSpec((blk,), lambda i: (i,)),
            pl.BlockSpec((blk,), lambda i: (i,)),
        ],
        out_shape=[
            jax.ShapeDtypeStruct((B,), jnp.float32),
            jax.ShapeDtypeStruct((B,), jnp.float32),
        ],
    )(sim_matrix)

    unc = pl.pallas_call(
        functools.partial(_tc2_body, k=k),
        grid=(grid,),
        in_specs=[
            pl.BlockSpec((blk, B), lambda i: (i, 0)),
            pl.BlockSpec((B, blk), lambda i: (0, i)),
            pl.BlockSpec((blk,), lambda i: (i,)),
            pl.BlockSpec((blk,), lambda i: (i,)),
            pl.BlockSpec((blk,), lambda i: (i,)),
        ],
        out_specs=pl.BlockSpec((blk,), lambda i: (i,)),
        out_shape=jax.ShapeDtypeStruct((B,), jnp.float32),
    )(sim_matrix, sim_matrix, t_row, t_col, ent)

    return (unc, ent)


# speedup vs baseline: 1.3550x; 1.3550x over previous
import functools

import jax
import jax.numpy as jnp
import numpy as np
from jax.experimental import pallas as pl
from jax.experimental.pallas import tpu as pltpu

_TEMPERATURE = 0.02
_K = 10
_PRE = 5
_NEG = float(np.finfo(np.float32).min)


def _fused_body(row_ref, col_ref, unc_ref, ent_ref, *, k: int, max_ent: float):
    X = row_ref[...]
    Y = col_ref[...]
    blk = X.shape[0]

    xm = X
    tr = None
    rowmax = None
    for it in range(k):
        tr = jnp.max(xm, axis=1, keepdims=True)
        if it == 0:
            rowmax = tr
        xm = jnp.where(xm >= tr, _NEG, xm)

    B = Y.shape[0]
    nch = B // 128
    cm = Y.reshape(nch, 128, blk)
    cands = []
    for _ in range(_PRE):
        t4 = jnp.max(cm, axis=1, keepdims=True)
        cm = jnp.where(cm >= t4, _NEG, cm)
        cands.append(t4.reshape(nch, blk))
    cand = jnp.concatenate(cands, axis=0)
    tcc = None
    for _ in range(k):
        tcc = jnp.max(cand, axis=0, keepdims=True)
        cand = jnp.where(cand >= tcc, _NEG, cand)
    cnt = jnp.sum((Y >= tcc).astype(jnp.float32), axis=0, keepdims=True)
    bad = jnp.any(cnt != float(k))

    def _full_col():
        ym = Y
        t = None
        for _ in range(k):
            t = jnp.max(ym, axis=0, keepdims=True)
            ym = jnp.where(ym >= t, _NEG, ym)
        return t

    tc = jax.lax.cond(bad, _full_col, lambda: tcc)

    inv_t = 1.0 / _TEMPERATURE
    sm = (X - rowmax) * inv_t
    e = jnp.exp(sm)
    Z = jnp.sum(e, axis=1, keepdims=True)
    S1 = jnp.sum(sm * e, axis=1, keepdims=True)
    ent = (jnp.log(Z) - S1 / Z)[:, 0] * (1.0 / max_ent)

    R = (X >= tr).astype(jnp.float32)
    C = (Y >= tc).astype(jnp.float32)
    P = jax.lax.dot(R, C, preferred_element_type=jnp.float32)
    ii = jax.lax.broadcasted_iota(jnp.int32, (blk, blk), 0)
    jj = jax.lax.broadcasted_iota(jnp.int32, (blk, blk), 1)
    matches = jnp.sum(jnp.where(ii == jj, P, 0.0), axis=1)

    ra = matches * (1.0 / k)
    unc_ref[...] = (1.0 - ra) * 0.5 + ent * 0.5
    ent_ref[...] = ent


def kernel(sim_matrix, pids):
    del pids
    B = sim_matrix.shape[0]
    blk = 256
    k = min(_K, B)
    max_ent = float(np.log(B + 1e-10))
    grid = B // blk
    unc, ent = pl.pallas_call(
        functools.partial(_fused_body, k=k, max_ent=max_ent),
        grid=(grid,),
        in_specs=[
            pl.BlockSpec((blk, B), lambda i: (i, 0)),
            pl.BlockSpec((B, blk), lambda i: (0, i)),
        ],
        out_specs=[
            pl.BlockSpec((blk,), lambda i: (i,)),
            pl.BlockSpec((blk,), lambda i: (i,)),
        ],
        out_shape=[
            jax.ShapeDtypeStruct((B,), jnp.float32),
            jax.ShapeDtypeStruct((B,), jnp.float32),
        ],
    )(sim_matrix, sim_matrix)
    return (unc, ent)
